# baseline (speedup 1.0000x reference)
import functools

import jax
import jax.numpy as jnp
from jax import lax
from jax.experimental import pallas as pl
from jax.experimental.pallas import tpu as pltpu
from jax.experimental.pallas import tpu_sc as plsc

D_MODEL = 768
D_FF = 2048
E = 8
T = 2048
LANES = 128
BM = 256
NB = T // BM
NT = NB + E - 1
CH = 128


def _routing_body(x_ref, wp_ref, bp_ref, dest_ref, pmax_ref, cnt_ref, sum_ref,
                  p16_ref):
    x = x_ref[...]
    logits = jnp.dot(x, wp_ref[...], preferred_element_type=jnp.float32)
    logits = logits + bp_ref[...]
    mx = jnp.max(logits, axis=1, keepdims=True)
    el = jnp.exp(logits - mx)
    s = jnp.sum(el, axis=1, keepdims=True)
    p = el / s
    pmx = jnp.max(p, axis=1, keepdims=True)
    pmax_ref[...] = pmx
    p16_ref[...] = jnp.broadcast_to(pmx, (T, 128))
    lane = lax.broadcasted_iota(jnp.int32, (T, LANES), 1)
    routes = jnp.min(jnp.where(logits == mx, lane, LANES), axis=1, keepdims=True)
    onehot = (lane == routes).astype(jnp.float32)
    sum_ref[...] = jnp.sum(p, axis=0, keepdims=True)

    r = lax.broadcasted_iota(jnp.int32, (CH, CH), 0)
    c = lax.broadcasted_iota(jnp.int32, (CH, CH), 1)
    lt = (r >= c).astype(jnp.float32)
    carry = jnp.zeros((1, LANES), jnp.float32)
    rank_parts = []
    lane_c = lax.broadcasted_iota(jnp.int32, (CH, LANES), 1)
    for k in range(T // CH):
        chunk = lax.slice(onehot, (k * CH, 0), ((k + 1) * CH, LANES))
        cs = jnp.dot(lt, chunk, preferred_element_type=jnp.float32) + carry
        routes_k = lax.slice(routes, (k * CH, 0), ((k + 1) * CH, 1))
        rank_parts.append(jnp.sum(jnp.where(lane_c == routes_k, cs, 0.0),
                                  axis=1, keepdims=True))
        carry = carry + jnp.sum(chunk, axis=0, keepdims=True)
    rank = jnp.concatenate(rank_parts, axis=0)
    counts = carry
    cnt_ref[...] = counts.astype(jnp.int32)
    rr = lax.broadcasted_iota(jnp.int32, (LANES, LANES), 0)
    cc = lax.broadcasted_iota(jnp.int32, (LANES, LANES), 1)
    upper = (rr < cc).astype(jnp.float32)
    offs = jnp.dot(counts, upper, preferred_element_type=jnp.float32,
                   precision=lax.Precision.HIGHEST)
    offs_tok = jnp.sum(jnp.where(lane == routes, offs, 0.0), axis=1, keepdims=True)
    dest_ref[...] = (rank - 1.0 + offs_tok).astype(jnp.int32)


def _run_routing(x2d, wp, bp):
    return pl.pallas_call(
        _routing_body,
        out_shape=[
            jax.ShapeDtypeStruct((T, 1), jnp.int32),
            jax.ShapeDtypeStruct((T, 1), jnp.float32),
            jax.ShapeDtypeStruct((1, LANES), jnp.int32),
            jax.ShapeDtypeStruct((1, LANES), jnp.float32),
            jax.ShapeDtypeStruct((T, 128), jnp.float32),
        ],
    )(x2d, wp, bp)


def _make_sc_dispatch():
    info = plsc.get_sparse_core_info()
    nw = info.num_cores * info.num_subcores
    tpw = T // nw
    mesh = plsc.VectorSubcoreMesh(core_axis_name="c", subcore_axis_name="s")

    @functools.partial(
        pl.kernel,
        out_type=[
            jax.ShapeDtypeStruct((T, D_MODEL), jnp.float32),
            jax.ShapeDtypeStruct((T, 128), jnp.float32),
        ],
        mesh=mesh,
        scratch_types=[
            pltpu.VMEM((tpw,), jnp.int32),
            pltpu.VMEM((tpw, D_MODEL), jnp.float32),
            pltpu.VMEM((tpw, 128), jnp.float32),
            pltpu.SemaphoreType.DMA,
            pltpu.SemaphoreType.DMA,
        ],
        compiler_params=pltpu.CompilerParams(use_tc_tiling_on_sc=True),
    )
    def body(x_hbm, p_hbm, idx_hbm, xs_hbm, ps_hbm,
             idx_v, rows_v, prow_v, sem1, sem2):
        wid = lax.axis_index("s") * info.num_cores + lax.axis_index("c")
        base = wid * tpw
        pltpu.sync_copy(idx_hbm.at[pl.ds(base, tpw)], idx_v)
        pltpu.sync_copy(x_hbm.at[pl.ds(base, tpw)], rows_v)
        pltpu.sync_copy(p_hbm.at[pl.ds(base, tpw)], prow_v)
        c1 = pltpu.async_copy(rows_v, xs_hbm.at[idx_v], sem1)
        c2 = pltpu.async_copy(prow_v, ps_hbm.at[idx_v], sem2)
        c1.wait()
        c2.wait()

    return body


def _make_sc_gather():
    info = plsc.get_sparse_core_info()
    nw = info.num_cores * info.num_subcores
    tpw = T // nw
    mesh = plsc.VectorSubcoreMesh(core_axis_name="c", subcore_axis_name="s")

    @functools.partial(
        pl.kernel,
        out_type=jax.ShapeDtypeStruct((T, 1, D_MODEL), jnp.float32),
        mesh=mesh,
        scratch_types=[
            pltpu.VMEM((tpw, 1, D_MODEL), jnp.float32),
            pltpu.VMEM((tpw,), jnp.int32),
            pltpu.SemaphoreType.DMA,
        ],
    )
    def body(src_hbm, idx_hbm, out_hbm, rows_v, idx_v, sem):
        wid = lax.axis_index("s") * info.num_cores + lax.axis_index("c")
        base = wid * tpw
        pltpu.sync_copy(idx_hbm.at[pl.ds(base, tpw)], idx_v)
        pltpu.async_copy(src_hbm.at[idx_v], rows_v, sem).wait()
        pltpu.sync_copy(rows_v, out_hbm.at[pl.ds(base, tpw)])

    return body


def _ffn_body(bid_ref, eid_ref, ts_ref, te_ref,
              x_ref, w1_ref, b1_ref, w2_ref, b2_ref, ps_ref, o_ref):
    i = pl.program_id(0)
    b = bid_ref[i]
    h = jnp.dot(x_ref[pl.ds(b * BM, BM), :], w1_ref[0],
                preferred_element_type=jnp.float32)
    h = jnp.maximum(h + b1_ref[0], 0.0)
    o = jnp.dot(h, w2_ref[0], preferred_element_type=jnp.float32) + b2_ref[0]
    o = o * ps_ref[...][:, 0:1]
    rows = b * BM + lax.broadcasted_iota(jnp.int32, (BM, 1), 0)
    mask = (rows >= ts_ref[i]) & (rows < te_ref[i])
    o = jnp.where(mask, o, 0.0)

    @pl.when(i == 0)
    def _():
        o_ref[...] = jnp.zeros_like(o_ref)

    o_ref[pl.ds(b * BM, BM), 0, :] += o


def _run_ffn(bid, eid, ts, te, xs, w1, b1, w2, b2, ps16):
    grid_spec = pltpu.PrefetchScalarGridSpec(
        num_scalar_prefetch=4,
        grid=(NT,),
        in_specs=[
            pl.BlockSpec((T, D_MODEL), lambda i, bid, eid, ts, te: (0, 0)),
            pl.BlockSpec((1, D_MODEL, D_FF),
                         lambda i, bid, eid, ts, te: (eid[i], 0, 0)),
            pl.BlockSpec((1, 1, D_FF),
                         lambda i, bid, eid, ts, te: (eid[i], 0, 0)),
            pl.BlockSpec((1, D_FF, D_MODEL),
                         lambda i, bid, eid, ts, te: (eid[i], 0, 0)),
            pl.BlockSpec((1, 1, D_MODEL),
                         lambda i, bid, eid, ts, te: (eid[i], 0, 0)),
            pl.BlockSpec((BM, 128), lambda i, bid, eid, ts, te: (bid[i], 0)),
        ],
        out_specs=pl.BlockSpec((T, 1, D_MODEL),
                               lambda i, bid, eid, ts, te: (0, 0, 0)),
    )
    return pl.pallas_call(
        _ffn_body,
        grid_spec=grid_spec,
        out_shape=jax.ShapeDtypeStruct((T, 1, D_MODEL), jnp.float32),
        compiler_params=pltpu.CompilerParams(
            dimension_semantics=("arbitrary",)),
    )(bid, eid, ts, te, xs, w1,
      b1.reshape(E, 1, D_FF), w2, b2.reshape(E, 1, D_MODEL), ps16)


def _tile_metadata(counts):
    ends = jnp.cumsum(counts)
    offs = ends - counts
    lo = jnp.arange(NB, dtype=jnp.int32) * BM
    ov = (offs[None, :] < (lo + BM)[:, None]) & (ends[None, :] > lo[:, None])
    flat = ov.reshape(-1)
    fi = jnp.arange(NB * E, dtype=jnp.int32)
    key = (fi % E) * NB + fi // E
    order = jnp.argsort(jnp.where(flat, key, NB * E + key))
    num_real = jnp.sum(flat.astype(jnp.int32))
    sel = order[jnp.minimum(jnp.arange(NT, dtype=jnp.int32), num_real - 1)]
    bid = (sel // E).astype(jnp.int32)
    eid = (sel % E).astype(jnp.int32)
    blo = bid * BM
    ts = jnp.maximum(offs[eid], blo).astype(jnp.int32)
    te = jnp.minimum(ends[eid], blo + BM).astype(jnp.int32)
    te = jnp.where(jnp.arange(NT, dtype=jnp.int32) >= num_real, ts, te)
    return bid, eid, ts, te


def kernel(x, switch_W, switch_b, W1, b1, W2, b2):
    seq_len, batch_size, d_model = x.shape
    x2d = x.reshape(T, D_MODEL)
    wp = jnp.zeros((D_MODEL, LANES), jnp.float32).at[:, :E].set(switch_W)
    bp = jnp.full((1, LANES), -1e30, jnp.float32).at[0, :E].set(switch_b)

    dest2d, pmax2d, cnt_pad, sum_pad, p16 = _run_routing(x2d, wp, bp)
    dest = dest2d.reshape(T)
    counts = cnt_pad[0, :E]

    xs, ps16 = _make_sc_dispatch()(x2d, p16, dest)
    bid, eid, ts, te = _tile_metadata(counts)
    ys = _run_ffn(bid, eid, ts, te, xs, W1, b1, W2, b2, ps16)
    final = _make_sc_gather()(ys, dest)

    route_prob_sums = sum_pad[0, :E]
    n_dropped = jnp.zeros((), jnp.int32)
    route_prob_max = pmax2d.reshape(T)
    return final, counts, route_prob_sums, n_dropped, route_prob_max


# speedup vs baseline: 1.0356x; 1.0356x over previous
import functools

import jax
import jax.numpy as jnp
from jax import lax
from jax.experimental import pallas as pl
from jax.experimental.pallas import tpu as pltpu
from jax.experimental.pallas import tpu_sc as plsc

D_MODEL = 768
D_FF = 2048
E = 8
T = 2048
LANES = 128
BM = 256
NB = T // BM
NT = NB + E - 1
CH = 128


def _routing_body(x_ref, wp_ref, bp_ref, dest_ref, pmax_ref, cnt_ref, sum_ref,
                  p16_ref):
    x = x_ref[...]
    logits = jnp.dot(x, wp_ref[...], preferred_element_type=jnp.float32)
    logits = logits + bp_ref[...]
    mx = jnp.max(logits, axis=1, keepdims=True)
    el = jnp.exp(logits - mx)
    s = jnp.sum(el, axis=1, keepdims=True)
    p = el / s
    pmx = jnp.max(p, axis=1, keepdims=True)
    pmax_ref[...] = pmx
    p16_ref[...] = jnp.broadcast_to(pmx, (T, 128))
    lane = lax.broadcasted_iota(jnp.int32, (T, LANES), 1)
    routes = jnp.min(jnp.where(logits == mx, lane, LANES), axis=1, keepdims=True)
    onehot = (lane == routes).astype(jnp.float32)
    sum_ref[...] = jnp.sum(p, axis=0, keepdims=True)

    r = lax.broadcasted_iota(jnp.int32, (CH, CH), 0)
    c = lax.broadcasted_iota(jnp.int32, (CH, CH), 1)
    lt = (r >= c).astype(jnp.float32)
    carry = jnp.zeros((1, LANES), jnp.float32)
    rank_parts = []
    lane_c = lax.broadcasted_iota(jnp.int32, (CH, LANES), 1)
    for k in range(T // CH):
        chunk = lax.slice(onehot, (k * CH, 0), ((k + 1) * CH, LANES))
        cs = jnp.dot(lt, chunk, preferred_element_type=jnp.float32) + carry
        routes_k = lax.slice(routes, (k * CH, 0), ((k + 1) * CH, 1))
        rank_parts.append(jnp.sum(jnp.where(lane_c == routes_k, cs, 0.0),
                                  axis=1, keepdims=True))
        carry = carry + jnp.sum(chunk, axis=0, keepdims=True)
    rank = jnp.concatenate(rank_parts, axis=0)
    counts = carry
    cnt_ref[...] = counts.astype(jnp.int32)
    rr = lax.broadcasted_iota(jnp.int32, (LANES, LANES), 0)
    cc = lax.broadcasted_iota(jnp.int32, (LANES, LANES), 1)
    upper = (rr < cc).astype(jnp.float32)
    offs = jnp.dot(counts, upper, preferred_element_type=jnp.float32,
                   precision=lax.Precision.HIGHEST)
    offs_tok = jnp.sum(jnp.where(lane == routes, offs, 0.0), axis=1, keepdims=True)
    dest_ref[...] = (rank - 1.0 + offs_tok).astype(jnp.int32)


def _run_routing(x2d, wp, bp):
    return pl.pallas_call(
        _routing_body,
        out_shape=[
            jax.ShapeDtypeStruct((T, 1), jnp.int32),
            jax.ShapeDtypeStruct((T, 1), jnp.float32),
            jax.ShapeDtypeStruct((1, LANES), jnp.int32),
            jax.ShapeDtypeStruct((1, LANES), jnp.float32),
            jax.ShapeDtypeStruct((T, 128), jnp.float32),
        ],
    )(x2d, wp, bp)


def _make_sc_dispatch():
    info = plsc.get_sparse_core_info()
    nw = info.num_cores * info.num_subcores
    tpw = T // nw
    mesh = plsc.VectorSubcoreMesh(core_axis_name="c", subcore_axis_name="s")

    @functools.partial(
        pl.kernel,
        out_type=[
            jax.ShapeDtypeStruct((T, D_MODEL), jnp.float32),
            jax.ShapeDtypeStruct((T, 128), jnp.float32),
        ],
        mesh=mesh,
        scratch_types=[
            pltpu.VMEM((tpw,), jnp.int32),
            pltpu.VMEM((tpw, D_MODEL), jnp.float32),
            pltpu.VMEM((tpw, 128), jnp.float32),
            pltpu.SemaphoreType.DMA,
            pltpu.SemaphoreType.DMA,
        ],
        compiler_params=pltpu.CompilerParams(use_tc_tiling_on_sc=True),
    )
    def body(x_hbm, p_hbm, idx_hbm, xs_hbm, ps_hbm,
             idx_v, rows_v, prow_v, sem1, sem2):
        wid = lax.axis_index("s") * info.num_cores + lax.axis_index("c")
        base = wid * tpw
        pltpu.sync_copy(idx_hbm.at[pl.ds(base, tpw)], idx_v)
        pltpu.sync_copy(x_hbm.at[pl.ds(base, tpw)], rows_v)
        pltpu.sync_copy(p_hbm.at[pl.ds(base, tpw)], prow_v)
        c1 = pltpu.async_copy(rows_v, xs_hbm.at[idx_v], sem1)
        c2 = pltpu.async_copy(prow_v, ps_hbm.at[idx_v], sem2)
        c1.wait()
        c2.wait()

    return body


def _make_sc_gather():
    info = plsc.get_sparse_core_info()
    nw = info.num_cores * info.num_subcores
    tpw = T // nw
    mesh = plsc.VectorSubcoreMesh(core_axis_name="c", subcore_axis_name="s")

    @functools.partial(
        pl.kernel,
        out_type=jax.ShapeDtypeStruct((T, 1, D_MODEL), jnp.float32),
        mesh=mesh,
        scratch_types=[
            pltpu.VMEM((tpw, 1, D_MODEL), jnp.float32),
            pltpu.VMEM((tpw,), jnp.int32),
            pltpu.SemaphoreType.DMA,
        ],
    )
    def body(src_hbm, idx_hbm, out_hbm, rows_v, idx_v, sem):
        wid = lax.axis_index("s") * info.num_cores + lax.axis_index("c")
        base = wid * tpw
        pltpu.sync_copy(idx_hbm.at[pl.ds(base, tpw)], idx_v)
        pltpu.async_copy(src_hbm.at[idx_v], rows_v, sem).wait()
        pltpu.sync_copy(rows_v, out_hbm.at[pl.ds(base, tpw)])

    return body


def _ffn_body(bid_ref, eid_ref, ts_ref, te_ref,
              x_ref, w1_ref, b1_ref, w2_ref, b2_ref, ps_ref, o_ref):
    i = pl.program_id(0)
    b = bid_ref[i]
    h = jnp.dot(x_ref[...], w1_ref[0], preferred_element_type=jnp.float32)
    h = jnp.maximum(h + b1_ref[0], 0.0)
    o = jnp.dot(h, w2_ref[0], preferred_element_type=jnp.float32) + b2_ref[0]
    o = o * ps_ref[...][:, 0:1]
    rows = b * BM + lax.broadcasted_iota(jnp.int32, (BM, 1), 0)
    mask = (rows >= ts_ref[i]) & (rows < te_ref[i])
    o = jnp.where(mask, o, 0.0)

    @pl.when(i == 0)
    def _():
        o_ref[...] = jnp.zeros_like(o_ref)

    o_ref[pl.ds(b * BM, BM), 0, :] += o


def _run_ffn(bid, eid, ts, te, xs, w1, b1, w2, b2, ps16):
    grid_spec = pltpu.PrefetchScalarGridSpec(
        num_scalar_prefetch=4,
        grid=(NT,),
        in_specs=[
            pl.BlockSpec((BM, D_MODEL), lambda i, bid, eid, ts, te: (bid[i], 0)),
            pl.BlockSpec((1, D_MODEL, D_FF),
                         lambda i, bid, eid, ts, te: (eid[i], 0, 0)),
            pl.BlockSpec((1, 1, D_FF),
                         lambda i, bid, eid, ts, te: (eid[i], 0, 0)),
            pl.BlockSpec((1, D_FF, D_MODEL),
                         lambda i, bid, eid, ts, te: (eid[i], 0, 0)),
            pl.BlockSpec((1, 1, D_MODEL),
                         lambda i, bid, eid, ts, te: (eid[i], 0, 0)),
            pl.BlockSpec((BM, 128), lambda i, bid, eid, ts, te: (bid[i], 0)),
        ],
        out_specs=pl.BlockSpec((T, 1, D_MODEL),
                               lambda i, bid, eid, ts, te: (0, 0, 0)),
    )
    return pl.pallas_call(
        _ffn_body,
        grid_spec=grid_spec,
        out_shape=jax.ShapeDtypeStruct((T, 1, D_MODEL), jnp.float32),
        compiler_params=pltpu.CompilerParams(
            dimension_semantics=("arbitrary",)),
    )(bid, eid, ts, te, xs, w1,
      b1.reshape(E, 1, D_FF), w2, b2.reshape(E, 1, D_MODEL), ps16)


def _tile_metadata(counts):
    ends = jnp.cumsum(counts)
    offs = ends - counts
    lo = jnp.arange(NB, dtype=jnp.int32) * BM
    ov = (offs[None, :] < (lo + BM)[:, None]) & (ends[None, :] > lo[:, None])
    flat = ov.reshape(-1)
    fi = jnp.arange(NB * E, dtype=jnp.int32)
    key = (fi % E) * NB + fi // E
    order = jnp.argsort(jnp.where(flat, key, NB * E + key))
    num_real = jnp.sum(flat.astype(jnp.int32))
    sel = order[jnp.minimum(jnp.arange(NT, dtype=jnp.int32), num_real - 1)]
    bid = (sel // E).astype(jnp.int32)
    eid = (sel % E).astype(jnp.int32)
    blo = bid * BM
    ts = jnp.maximum(offs[eid], blo).astype(jnp.int32)
    te = jnp.minimum(ends[eid], blo + BM).astype(jnp.int32)
    te = jnp.where(jnp.arange(NT, dtype=jnp.int32) >= num_real, ts, te)
    return bid, eid, ts, te


def kernel(x, switch_W, switch_b, W1, b1, W2, b2):
    seq_len, batch_size, d_model = x.shape
    x2d = x.reshape(T, D_MODEL)
    wp = jnp.zeros((D_MODEL, LANES), jnp.float32).at[:, :E].set(switch_W)
    bp = jnp.full((1, LANES), -1e30, jnp.float32).at[0, :E].set(switch_b)

    dest2d, pmax2d, cnt_pad, sum_pad, p16 = _run_routing(x2d, wp, bp)
    dest = dest2d.reshape(T)
    counts = cnt_pad[0, :E]

    xs, ps16 = _make_sc_dispatch()(x2d, p16, dest)
    bid, eid, ts, te = _tile_metadata(counts)
    ys = _run_ffn(bid, eid, ts, te, xs, W1, b1, W2, b2, ps16)
    final = _make_sc_gather()(ys, dest)

    route_prob_sums = sum_pad[0, :E]
    n_dropped = jnp.zeros((), jnp.int32)
    route_prob_max = pmax2d.reshape(T)
    return final, counts, route_prob_sums, n_dropped, route_prob_max


# speedup vs baseline: 1.0356x; 1.0000x over previous
import functools

import jax
import jax.numpy as jnp
from jax import lax
from jax.experimental import pallas as pl
from jax.experimental.pallas import tpu as pltpu
from jax.experimental.pallas import tpu_sc as plsc

D_MODEL = 768
D_FF = 2048
E = 8
T = 2048
LANES = 128
BM = 256
NB = T // BM
NT = NB + E - 1
CH = 128


def _routing_body(x_ref, wp_ref, bp_ref, dest_ref, pmax_ref, cnt_ref, sum_ref,
                  p16_ref):
    x = x_ref[...]
    logits = jnp.dot(x, wp_ref[...], preferred_element_type=jnp.float32)
    logits = logits + bp_ref[...]
    mx = jnp.max(logits, axis=1, keepdims=True)
    el = jnp.exp(logits - mx)
    s = jnp.sum(el, axis=1, keepdims=True)
    p = el / s
    pmx = jnp.max(p, axis=1, keepdims=True)
    pmax_ref[...] = pmx
    p16_ref[...] = jnp.broadcast_to(pmx, (T, 128))
    lane = lax.broadcasted_iota(jnp.int32, (T, LANES), 1)
    routes = jnp.min(jnp.where(logits == mx, lane, LANES), axis=1, keepdims=True)
    onehot = (lane == routes).astype(jnp.float32)
    sum_ref[...] = jnp.sum(p, axis=0, keepdims=True)

    r = lax.broadcasted_iota(jnp.int32, (CH, CH), 0)
    c = lax.broadcasted_iota(jnp.int32, (CH, CH), 1)
    lt = (r >= c).astype(jnp.float32)
    carry = jnp.zeros((1, LANES), jnp.float32)
    rank_parts = []
    lane_c = lax.broadcasted_iota(jnp.int32, (CH, LANES), 1)
    for k in range(T // CH):
        chunk = lax.slice(onehot, (k * CH, 0), ((k + 1) * CH, LANES))
        cs = jnp.dot(lt, chunk, preferred_element_type=jnp.float32) + carry
        routes_k = lax.slice(routes, (k * CH, 0), ((k + 1) * CH, 1))
        rank_parts.append(jnp.sum(jnp.where(lane_c == routes_k, cs, 0.0),
                                  axis=1, keepdims=True))
        carry = carry + jnp.sum(chunk, axis=0, keepdims=True)
    rank = jnp.concatenate(rank_parts, axis=0)
    counts = carry
    cnt_ref[...] = counts.astype(jnp.int32)
    rr = lax.broadcasted_iota(jnp.int32, (LANES, LANES), 0)
    cc = lax.broadcasted_iota(jnp.int32, (LANES, LANES), 1)
    upper = (rr < cc).astype(jnp.float32)
    offs = jnp.dot(counts, upper, preferred_element_type=jnp.float32,
                   precision=lax.Precision.HIGHEST)
    offs_tok = jnp.sum(jnp.where(lane == routes, offs, 0.0), axis=1, keepdims=True)
    dest_ref[...] = (rank - 1.0 + offs_tok).astype(jnp.int32)


def _run_routing(x2d, wp, bp):
    return pl.pallas_call(
        _routing_body,
        out_shape=[
            jax.ShapeDtypeStruct((T, 1), jnp.int32),
            jax.ShapeDtypeStruct((T, 1), jnp.float32),
            jax.ShapeDtypeStruct((1, LANES), jnp.int32),
            jax.ShapeDtypeStruct((1, LANES), jnp.float32),
            jax.ShapeDtypeStruct((T, 128), jnp.float32),
        ],
    )(x2d, wp, bp)


def _make_sc_dispatch():
    info = plsc.get_sparse_core_info()
    nw = info.num_cores * info.num_subcores
    tpw = T // nw
    mesh = plsc.VectorSubcoreMesh(core_axis_name="c", subcore_axis_name="s")

    @functools.partial(
        pl.kernel,
        out_type=[
            jax.ShapeDtypeStruct((T, D_MODEL), jnp.float32),
            jax.ShapeDtypeStruct((T, 128), jnp.float32),
        ],
        mesh=mesh,
        scratch_types=[
            pltpu.VMEM((tpw,), jnp.int32),
            pltpu.VMEM((tpw, D_MODEL), jnp.float32),
            pltpu.VMEM((tpw, 128), jnp.float32),
            pltpu.SemaphoreType.DMA,
            pltpu.SemaphoreType.DMA,
        ],
    )
    def body(x_hbm, p_hbm, idx_hbm, xs_hbm, ps_hbm,
             idx_v, rows_v, prow_v, sem1, sem2):
        wid = lax.axis_index("s") * info.num_cores + lax.axis_index("c")
        base = wid * tpw
        pltpu.sync_copy(idx_hbm.at[pl.ds(base, tpw)], idx_v)
        pltpu.sync_copy(x_hbm.at[pl.ds(base, tpw)], rows_v)
        pltpu.sync_copy(p_hbm.at[pl.ds(base, tpw)], prow_v)
        c1 = pltpu.async_copy(rows_v, xs_hbm.at[idx_v], sem1)
        c2 = pltpu.async_copy(prow_v, ps_hbm.at[idx_v], sem2)
        c1.wait()
        c2.wait()

    return body


def _make_sc_gather():
    info = plsc.get_sparse_core_info()
    nw = info.num_cores * info.num_subcores
    tpw = T // nw
    mesh = plsc.VectorSubcoreMesh(core_axis_name="c", subcore_axis_name="s")

    @functools.partial(
        pl.kernel,
        out_type=jax.ShapeDtypeStruct((T, 1, D_MODEL), jnp.float32),
        mesh=mesh,
        scratch_types=[
            pltpu.VMEM((tpw, 1, D_MODEL), jnp.float32),
            pltpu.VMEM((tpw,), jnp.int32),
            pltpu.SemaphoreType.DMA,
        ],
    )
    def body(src_hbm, idx_hbm, out_hbm, rows_v, idx_v, sem):
        wid = lax.axis_index("s") * info.num_cores + lax.axis_index("c")
        base = wid * tpw
        pltpu.sync_copy(idx_hbm.at[pl.ds(base, tpw)], idx_v)
        pltpu.async_copy(src_hbm.at[idx_v], rows_v, sem).wait()
        pltpu.sync_copy(rows_v, out_hbm.at[pl.ds(base, tpw)])

    return body


def _ffn_body(bid_ref, eid_ref, ts_ref, te_ref,
              x_ref, w1_ref, b1_ref, w2_ref, b2_ref, ps_ref, o_ref):
    i = pl.program_id(0)
    b = bid_ref[i]
    h = jnp.dot(x_ref[...], w1_ref[0], preferred_element_type=jnp.float32)
    h = jnp.maximum(h + b1_ref[0], 0.0)
    o = jnp.dot(h, w2_ref[0], preferred_element_type=jnp.float32) + b2_ref[0]
    o = o * ps_ref[...][:, 0:1]
    rows = b * BM + lax.broadcasted_iota(jnp.int32, (BM, 1), 0)
    mask = (rows >= ts_ref[i]) & (rows < te_ref[i])
    o = jnp.where(mask, o, 0.0)

    @pl.when(i == 0)
    def _():
        o_ref[...] = jnp.zeros_like(o_ref)

    o_ref[pl.ds(b * BM, BM), 0, :] += o


def _run_ffn(bid, eid, ts, te, xs, w1, b1, w2, b2, ps16):
    grid_spec = pltpu.PrefetchScalarGridSpec(
        num_scalar_prefetch=4,
        grid=(NT,),
        in_specs=[
            pl.BlockSpec((BM, D_MODEL), lambda i, bid, eid, ts, te: (bid[i], 0)),
            pl.BlockSpec((1, D_MODEL, D_FF),
                         lambda i, bid, eid, ts, te: (eid[i], 0, 0)),
            pl.BlockSpec((1, 1, D_FF),
                         lambda i, bid, eid, ts, te: (eid[i], 0, 0)),
            pl.BlockSpec((1, D_FF, D_MODEL),
                         lambda i, bid, eid, ts, te: (eid[i], 0, 0)),
            pl.BlockSpec((1, 1, D_MODEL),
                         lambda i, bid, eid, ts, te: (eid[i], 0, 0)),
            pl.BlockSpec((BM, 128), lambda i, bid, eid, ts, te: (bid[i], 0)),
        ],
        out_specs=pl.BlockSpec((T, 1, D_MODEL),
                               lambda i, bid, eid, ts, te: (0, 0, 0)),
    )
    return pl.pallas_call(
        _ffn_body,
        grid_spec=grid_spec,
        out_shape=jax.ShapeDtypeStruct((T, 1, D_MODEL), jnp.float32),
        compiler_params=pltpu.CompilerParams(
            dimension_semantics=("arbitrary",)),
    )(bid, eid, ts, te, xs, w1,
      b1.reshape(E, 1, D_FF), w2, b2.reshape(E, 1, D_MODEL), ps16)


def _tile_metadata(counts):
    ends = jnp.cumsum(counts)
    offs = ends - counts
    lo = jnp.arange(NB, dtype=jnp.int32) * BM
    ov = (offs[None, :] < (lo + BM)[:, None]) & (ends[None, :] > lo[:, None])
    flat = ov.reshape(-1)
    fi = jnp.arange(NB * E, dtype=jnp.int32)
    key = (fi % E) * NB + fi // E
    order = jnp.argsort(jnp.where(flat, key, NB * E + key))
    num_real = jnp.sum(flat.astype(jnp.int32))
    sel = order[jnp.minimum(jnp.arange(NT, dtype=jnp.int32), num_real - 1)]
    bid = (sel // E).astype(jnp.int32)
    eid = (sel % E).astype(jnp.int32)
    blo = bid * BM
    ts = jnp.maximum(offs[eid], blo).astype(jnp.int32)
    te = jnp.minimum(ends[eid], blo + BM).astype(jnp.int32)
    te = jnp.where(jnp.arange(NT, dtype=jnp.int32) >= num_real, ts, te)
    return bid, eid, ts, te


def kernel(x, switch_W, switch_b, W1, b1, W2, b2):
    seq_len, batch_size, d_model = x.shape
    x2d = x.reshape(T, D_MODEL)
    wp = jnp.zeros((D_MODEL, LANES), jnp.float32).at[:, :E].set(switch_W)
    bp = jnp.full((1, LANES), -1e30, jnp.float32).at[0, :E].set(switch_b)

    dest2d, pmax2d, cnt_pad, sum_pad, p16 = _run_routing(x2d, wp, bp)
    dest = dest2d.reshape(T)
    counts = cnt_pad[0, :E]

    xs, ps16 = _make_sc_dispatch()(x2d, p16, dest)
    bid, eid, ts, te = _tile_metadata(counts)
    ys = _run_ffn(bid, eid, ts, te, xs, W1, b1, W2, b2, ps16)
    final = _make_sc_gather()(ys, dest)

    route_prob_sums = sum_pad[0, :E]
    n_dropped = jnp.zeros((), jnp.int32)
    route_prob_max = pmax2d.reshape(T)
    return final, counts, route_prob_sums, n_dropped, route_prob_max
